# Initial kernel scaffold; baseline (speedup 1.0000x reference)
import jax
import jax.numpy as jnp
from jax.experimental import pallas as pl


def kernel(token_ids, weight):
    raise NotImplementedError("write your pallas kernel here")


# speedup vs baseline: 1.8754x; 1.8754x over previous
import functools

import jax
import jax.numpy as jnp
from jax import lax
from jax.experimental import pallas as pl
from jax.experimental.pallas import tpu as pltpu
from jax.experimental.pallas import tpu_sc as plsc

D = 64
CHUNK = 128
NBUF = 4


@functools.lru_cache(maxsize=None)
def _build(n_workers: int, n_chunks: int, vocab: int):
    mesh = plsc.VectorSubcoreMesh(core_axis_name="c", subcore_axis_name="s")

    @functools.partial(
        pl.kernel,
        mesh=mesh,
        out_type=jax.ShapeDtypeStruct((n_workers, n_chunks, CHUNK, D),
                                      jnp.float32),
        scratch_types=[
            pltpu.VMEM((n_chunks, CHUNK), jnp.int32),
            pltpu.VMEM((NBUF, CHUNK, D), jnp.float32),
        ] + [pltpu.SemaphoreType.DMA] * NBUF,
        compiler_params=pltpu.CompilerParams(use_tc_tiling_on_sc=False),
    )
    def k(idx_hbm, weight_hbm, out_hbm, idx_v, rows_v, *gsems):
        nc = plsc.get_sparse_core_info().num_cores
        wid = lax.axis_index("s") * nc + lax.axis_index("c")
        pltpu.sync_copy(idx_hbm.at[wid], idx_v)

        for b in range(NBUF):
            pltpu.async_copy(weight_hbm.at[idx_v.at[b]], rows_v.at[b],
                             gsems[b])

        def step(s, carry):
            for b in range(NBUF):
                g = s * NBUF + b
                pltpu.make_async_copy(weight_hbm.at[idx_v.at[g]],
                                      rows_v.at[b], gsems[b]).wait()
                pltpu.sync_copy(rows_v.at[b], out_hbm.at[wid, g])
                gn = jnp.minimum(g + NBUF, n_chunks - 1)
                pltpu.async_copy(weight_hbm.at[idx_v.at[gn]], rows_v.at[b],
                                 gsems[b])
            return carry

        lax.fori_loop(0, n_chunks // NBUF, step, 0)

        for b in range(NBUF):
            pltpu.make_async_copy(weight_hbm.at[idx_v.at[n_chunks - 1]],
                                  rows_v.at[b], gsems[b]).wait()

    return k


def kernel(token_ids, weight):
    batch, seq = token_ids.shape
    vocab, d = weight.shape
    total = batch * seq
    info = plsc.get_sparse_core_info()
    n_workers = info.num_cores * info.num_subcores
    n_chunks = total // (n_workers * CHUNK)
    idx = token_ids.reshape(n_workers, n_chunks, CHUNK).astype(jnp.int32)
    out = _build(n_workers, n_chunks, vocab)(idx, weight)
    return out.reshape(batch, seq, d)
